# baseline (speedup 1.0000x reference)
import functools

import jax
import jax.numpy as jnp
from jax import lax
from jax.experimental import pallas as pl
from jax.experimental.pallas import tpu as pltpu
from jax.experimental.pallas import tpu_sc as plsc

BATCH = 64
HEIGHT = 32
WIDTH = 32
EMBED_DIM = 256
MAX_POS = 50

SC_BATCH = 22
TC_BATCH = BATCH - SC_BATCH
BLOCK_B = 2

_MESH = plsc.VectorSubcoreMesh(core_axis_name="c", subcore_axis_name="s")


def _sc_body(row_hbm, col_hbm, out_hbm, chunk_v, sem):
    c = lax.axis_index("c")
    s = lax.axis_index("s")
    h = s * 2 + c
    pltpu.sync_copy(col_hbm.at[pl.ds(0, WIDTH), :], chunk_v.at[:, pl.ds(0, EMBED_DIM)])
    for w in range(WIDTH):
        pltpu.sync_copy(row_hbm.at[h], chunk_v.at[w, pl.ds(EMBED_DIM, EMBED_DIM)])
    copies = []
    for b in range(SC_BATCH):
        copies.append(
            pltpu.async_copy(
                chunk_v, out_hbm.at[b, pl.ds(h * WIDTH, WIDTH), :], sem
            )
        )
    for cp in copies:
        cp.wait()


_sc_kernel = functools.partial(
    pl.kernel,
    mesh=_MESH,
    out_type=jax.ShapeDtypeStruct(
        (SC_BATCH, HEIGHT * WIDTH, 2 * EMBED_DIM), jnp.float32
    ),
    scratch_types=[
        pltpu.VMEM((WIDTH, 2 * EMBED_DIM), jnp.float32),
        pltpu.SemaphoreType.DMA,
    ],
)(_sc_body)


def _tc_body(row_ref, col_ref, out_ref):
    x = col_ref[:WIDTH, :]
    y = row_ref[:HEIGHT, :]
    left = jnp.broadcast_to(x[None, :, :], (HEIGHT, WIDTH, EMBED_DIM))
    left = left.reshape(HEIGHT * WIDTH, EMBED_DIM)
    right = jnp.broadcast_to(y[:, None, :], (HEIGHT, WIDTH, EMBED_DIM))
    right = right.reshape(HEIGHT * WIDTH, EMBED_DIM)
    tile = jnp.concatenate([left, right], axis=-1)
    out_ref[...] = jnp.broadcast_to(
        tile[None], (BLOCK_B, HEIGHT * WIDTH, 2 * EMBED_DIM)
    )


def _tc_kernel(row_embeddings, column_embeddings):
    return pl.pallas_call(
        _tc_body,
        grid=(TC_BATCH // BLOCK_B,),
        in_specs=[
            pl.BlockSpec((MAX_POS, EMBED_DIM), lambda b: (0, 0)),
            pl.BlockSpec((MAX_POS, EMBED_DIM), lambda b: (0, 0)),
        ],
        out_specs=pl.BlockSpec(
            (BLOCK_B, HEIGHT * WIDTH, 2 * EMBED_DIM), lambda b: (b, 0, 0)
        ),
        out_shape=jax.ShapeDtypeStruct(
            (TC_BATCH, HEIGHT * WIDTH, 2 * EMBED_DIM), jnp.float32
        ),
        compiler_params=pltpu.CompilerParams(
            dimension_semantics=("arbitrary",),
        ),
    )(row_embeddings, column_embeddings)


def kernel(row_embeddings, column_embeddings):
    sc_out = _sc_kernel(row_embeddings, column_embeddings)
    tc_out = _tc_kernel(row_embeddings, column_embeddings)
    return jnp.concatenate([sc_out, tc_out], axis=0)


# speedup vs baseline: 3.6806x; 3.6806x over previous
import jax
import jax.numpy as jnp
from jax.experimental import pallas as pl
from jax.experimental.pallas import tpu as pltpu

BATCH = 64
HEIGHT = 32
WIDTH = 32
EMBED_DIM = 256
MAX_POS = 50

BLOCK_B = 8


def _body(row_ref, col_ref, out_ref):
    x = col_ref[:WIDTH, :]
    y = row_ref[:HEIGHT, :]
    left = jnp.broadcast_to(x[None, :, :], (HEIGHT, WIDTH, EMBED_DIM))
    left = left.reshape(HEIGHT * WIDTH, EMBED_DIM)
    right = jnp.broadcast_to(y[:, None, :], (HEIGHT, WIDTH, EMBED_DIM))
    right = right.reshape(HEIGHT * WIDTH, EMBED_DIM)
    tile = jnp.concatenate([left, right], axis=-1)
    out_ref[...] = jnp.broadcast_to(tile[None], (BLOCK_B, HEIGHT * WIDTH, 2 * EMBED_DIM))


def kernel(row_embeddings, column_embeddings):
    out = pl.pallas_call(
        _body,
        grid=(BATCH // BLOCK_B,),
        in_specs=[
            pl.BlockSpec((MAX_POS, EMBED_DIM), lambda b: (0, 0)),
            pl.BlockSpec((MAX_POS, EMBED_DIM), lambda b: (0, 0)),
        ],
        out_specs=pl.BlockSpec(
            (BLOCK_B, HEIGHT * WIDTH, 2 * EMBED_DIM), lambda b: (b, 0, 0)
        ),
        out_shape=jax.ShapeDtypeStruct(
            (BATCH, HEIGHT * WIDTH, 2 * EMBED_DIM), jnp.float32
        ),
        compiler_params=pltpu.CompilerParams(
            dimension_semantics=("arbitrary",),
        ),
    )(row_embeddings, column_embeddings)
    return out


# speedup vs baseline: 4.0268x; 1.0940x over previous
import jax
import jax.numpy as jnp
from jax.experimental import pallas as pl
from jax.experimental.pallas import tpu as pltpu

BATCH = 64
HEIGHT = 32
WIDTH = 32
EMBED_DIM = 256
MAX_POS = 50

BLOCK_B = 2


def _body(row_ref, col_ref, out_ref):
    x = col_ref[:WIDTH, :]
    y = row_ref[:HEIGHT, :]
    left = jnp.broadcast_to(x[None, :, :], (HEIGHT, WIDTH, EMBED_DIM))
    left = left.reshape(HEIGHT * WIDTH, EMBED_DIM)
    right = jnp.broadcast_to(y[:, None, :], (HEIGHT, WIDTH, EMBED_DIM))
    right = right.reshape(HEIGHT * WIDTH, EMBED_DIM)
    tile = jnp.concatenate([left, right], axis=-1)
    out_ref[...] = jnp.broadcast_to(tile[None], (BLOCK_B, HEIGHT * WIDTH, 2 * EMBED_DIM))


def kernel(row_embeddings, column_embeddings):
    out = pl.pallas_call(
        _body,
        grid=(BATCH // BLOCK_B,),
        in_specs=[
            pl.BlockSpec((MAX_POS, EMBED_DIM), lambda b: (0, 0)),
            pl.BlockSpec((MAX_POS, EMBED_DIM), lambda b: (0, 0)),
        ],
        out_specs=pl.BlockSpec(
            (BLOCK_B, HEIGHT * WIDTH, 2 * EMBED_DIM), lambda b: (b, 0, 0)
        ),
        out_shape=jax.ShapeDtypeStruct(
            (BATCH, HEIGHT * WIDTH, 2 * EMBED_DIM), jnp.float32
        ),
        compiler_params=pltpu.CompilerParams(
            dimension_semantics=("arbitrary",),
        ),
    )(row_embeddings, column_embeddings)
    return out
